# baseline (speedup 1.0000x reference)
import functools

import jax
import jax.numpy as jnp
from jax import lax
from jax.experimental import pallas as pl
from jax.experimental.pallas import tpu as pltpu
from jax.experimental.pallas import tpu_sc as plsc

N = 10000
E = 320000
D = 128
NC = 2
NS = 16
NW = NC * NS
CH = 128
EPAD = 327680
ROWS = EPAD // CH
RW = ROWS // NW
CHG = 64
NB = 4
CPT = RW * 2
G = CPT // NB
NP = 10240
RPT = NP // NS


def _sc_agg_body(with_counts, *refs):
    if with_counts:
        (h_hbm, src_hbm, dst_hbm, zeros_hbm, zeros1_hbm, ones1_hbm,
         out_hbm, cnt_hbm,
         src_t, r0b, r1b, r2b, r3b, d0b, d1b, d2b, d3b, ones_v,
         acc_sh, cnt_sh,
         g0, g1, g2, g3, f0, f1, f2, f3, s0, s1, s2, s3,
         c0s, c1s, c2s, c3s) = refs
    else:
        (h_hbm, src_hbm, dst_hbm, zeros_hbm,
         out_hbm,
         src_t, r0b, r1b, r2b, r3b, d0b, d1b, d2b, d3b,
         acc_sh,
         g0, g1, g2, g3, f0, f1, f2, f3, s0, s1, s2, s3) = refs
    rows = [r0b, r1b, r2b, r3b]
    dstv = [d0b, d1b, d2b, d3b]
    gsem = [g0, g1, g2, g3]
    fsem = [f0, f1, f2, f3]
    ssem = [s0, s1, s2, s3]
    if with_counts:
        csem = [c0s, c1s, c2s, c3s]

    c = lax.axis_index("c")
    s = lax.axis_index("s")
    wid = s * NC + c

    t0 = s * RPT
    pltpu.sync_copy(zeros_hbm, acc_sh.at[pl.ds(t0, RPT)])
    if with_counts:
        pltpu.sync_copy(zeros1_hbm, cnt_sh.at[pl.ds(t0, RPT)])
        pltpu.sync_copy(ones1_hbm, ones_v)
    plsc.subcore_barrier()

    base = wid * RW
    cbase = wid * CPT

    pltpu.sync_copy(src_hbm.at[pl.ds(base, RW)], src_t)

    def fire(slot, row, half, chunk_off):
        off = pl.multiple_of(chunk_off * CHG, CHG)
        pltpu.async_copy(dst_hbm.at[pl.ds(off, CHG)], dstv[slot], fsem[slot])
        pltpu.async_copy(h_hbm.at[src_t.at[row, pl.ds(half * CHG, CHG)]],
                         rows[slot], gsem[slot])

    def wait_scats(slot):
        pltpu.make_async_copy(rows[slot], acc_sh.at[pl.ds(0, CHG)],
                              ssem[slot]).wait()
        if with_counts:
            pltpu.make_async_copy(ones_v, cnt_sh.at[pl.ds(0, CHG)],
                                  csem[slot]).wait()

    fire(0, 0, 0, cbase)
    fire(1, 0, 1, cbase + 1)

    def grp(g, carry):
        for b in range(NB):
            pltpu.make_async_copy(h_hbm.at[pl.ds(0, CHG)], rows[b],
                                  gsem[b]).wait()
            pltpu.make_async_copy(dst_hbm.at[pl.ds(0, CHG)], dstv[b],
                                  fsem[b]).wait()
            pltpu.async_copy(rows[b], acc_sh.at[dstv[b]], ssem[b], add=True)
            if with_counts:
                pltpu.async_copy(ones_v, cnt_sh.at[dstv[b]], csem[b],
                                 add=True)
            if b == 0:
                @pl.when(g > 0)
                def _():
                    wait_scats(2)
                fire(2, 2 * g + 1, 0, cbase + 4 * g + 2)
            elif b == 1:
                @pl.when(g > 0)
                def _():
                    wait_scats(3)
                fire(3, 2 * g + 1, 1, cbase + 4 * g + 3)
            elif b == 2:
                wait_scats(0)

                @pl.when(g < G - 1)
                def _():
                    fire(0, 2 * g + 2, 0, cbase + 4 * g + 4)
            else:
                wait_scats(1)

                @pl.when(g < G - 1)
                def _():
                    fire(1, 2 * g + 2, 1, cbase + 4 * g + 5)
        return carry

    lax.fori_loop(0, G, grp, 0)

    wait_scats(2)
    wait_scats(3)

    plsc.subcore_barrier()

    pltpu.sync_copy(acc_sh.at[pl.ds(t0, RPT)], out_hbm.at[c, pl.ds(t0, RPT)])
    if with_counts:
        pltpu.sync_copy(cnt_sh.at[pl.ds(t0, RPT)],
                        cnt_hbm.at[pl.ds(c * NP + t0, RPT)])


def _make_sc_agg(with_counts):
    mesh = plsc.VectorSubcoreMesh(core_axis_name="c", subcore_axis_name="s")
    out_type = [jax.ShapeDtypeStruct((NC, NP, D), jnp.float32)]
    if with_counts:
        out_type.append(jax.ShapeDtypeStruct((NC * NP,), jnp.float32))
    scratch = [pltpu.VMEM((RW, CH), jnp.int32)]
    scratch += [pltpu.VMEM((CHG, D), jnp.float32)] * NB
    scratch += [pltpu.VMEM((CHG,), jnp.int32)] * NB
    if with_counts:
        scratch += [pltpu.VMEM((CHG,), jnp.float32)]
    scratch += [pltpu.VMEM_SHARED((NP, D), jnp.float32)]
    if with_counts:
        scratch += [pltpu.VMEM_SHARED((NP,), jnp.float32)]
    nsem = 4 * NB if with_counts else 3 * NB
    scratch += [pltpu.SemaphoreType.DMA] * nsem
    return pl.kernel(
        functools.partial(_sc_agg_body, with_counts),
        out_type=tuple(out_type) if with_counts else out_type[0],
        mesh=mesh,
        scratch_types=scratch,
    )


def _fastkan_block(mean, swa, swb, bwt, bias):
    ta = (mean + 1.0) * 0.5
    tb = (mean - 1.0) * 0.5
    ba = jnp.exp(-(ta * ta))
    bb = jnp.exp(-(tb * tb))
    sil = mean / (1.0 + jnp.exp(-mean))
    h = jnp.dot(ba, swa, preferred_element_type=jnp.float32)
    h += jnp.dot(bb, swb, preferred_element_type=jnp.float32)
    h += jnp.dot(sil, bwt, preferred_element_type=jnp.float32)
    return h + bias


def _layernorm_block(h):
    mu = jnp.mean(h, axis=-1, keepdims=True)
    var = jnp.mean((h - mu) ** 2, axis=-1, keepdims=True)
    return (h - mu) * lax.rsqrt(var + 1e-5)


def _tc_layer1_body(p0, p1, c0, c1, x, swa, swb, bwt, bias, out):
    agg = p0[...] + p1[...] + x[...]
    cnt = c0[...] + c1[...] + 1.0
    mean = agg / cnt
    h = _fastkan_block(mean, swa[...], swb[...], bwt[...], bias[...])
    out[...] = _layernorm_block(h)


def _tc_layer23_body(p0, p1, c0, c1, h1, swa2, swb2, bwt2, b2,
                     swa3, swb3, bwt3, b3, out):
    agg = p0[...] + p1[...] + h1[...]
    cnt = c0[...] + c1[...] + 1.0
    mean = agg / cnt
    h2 = _layernorm_block(
        _fastkan_block(mean, swa2[...], swb2[...], bwt2[...], b2[...]))
    o = _fastkan_block(h2, swa3[...], swb3[...], bwt3[...], b3[...])
    m = jnp.max(o, axis=-1, keepdims=True)
    lse = m + jnp.log(jnp.sum(jnp.exp(o - m), axis=-1, keepdims=True))
    out[...] = o - lse


_R = 1000


def _row_spec():
    return pl.BlockSpec((_R, D), lambda i: (i, 0))


def _cnt_spec():
    return pl.BlockSpec((_R, 1), lambda i: (i, 0))


def _w_spec():
    return pl.BlockSpec((D, D), lambda i: (0, 0))


def _b_spec():
    return pl.BlockSpec((1, D), lambda i: (0, 0))


def _split_w(sw, bw, sb, bb):
    swa = jnp.transpose(sw[:, 0::2])
    swb = jnp.transpose(sw[:, 1::2])
    bwt = jnp.transpose(bw)
    bias = (sb + bb).reshape(1, D)
    return swa, swb, bwt, bias


def kernel(x, edge_index, sw1, sb1, bw1, bb1, sw2, sb2, bw2, bb2,
           sw3, sb3, bw3, bb3):
    pad = EPAD - E
    pad_ar = jnp.arange(pad, dtype=jnp.int32)
    srcp = jnp.concatenate(
        [edge_index[0], pad_ar % N]).reshape(ROWS, CH)
    dstp = jnp.concatenate(
        [edge_index[1], N + pad_ar % (NP - N)])
    zeros = jnp.zeros((RPT, D), jnp.float32)
    zeros1 = jnp.zeros((RPT,), jnp.float32)
    ones1 = jnp.ones((CHG,), jnp.float32)

    w1 = _split_w(sw1, bw1, sb1, bb1)
    w2 = _split_w(sw2, bw2, sb2, bb2)
    w3 = _split_w(sw3, bw3, sb3, bb3)

    agg1, cnt1d = _make_sc_agg(True)(x, srcp, dstp, zeros, zeros1, ones1)
    c0 = cnt1d[:N].reshape(N, 1)
    c1 = cnt1d[NP:NP + N].reshape(N, 1)

    tc1 = pl.pallas_call(
        _tc_layer1_body,
        grid=(N // _R,),
        in_specs=[_row_spec(), _row_spec(), _cnt_spec(), _cnt_spec(),
                  _row_spec(), _w_spec(), _w_spec(), _w_spec(), _b_spec()],
        out_specs=_row_spec(),
        out_shape=jax.ShapeDtypeStruct((N, D), jnp.float32),
    )
    h1 = tc1(agg1[0], agg1[1], c0, c1, x, *w1)

    agg2 = _make_sc_agg(False)(h1, srcp, dstp, zeros)

    tc23 = pl.pallas_call(
        _tc_layer23_body,
        grid=(N // _R,),
        in_specs=[_row_spec(), _row_spec(), _cnt_spec(), _cnt_spec(),
                  _row_spec(),
                  _w_spec(), _w_spec(), _w_spec(), _b_spec(),
                  _w_spec(), _w_spec(), _w_spec(), _b_spec()],
        out_specs=_row_spec(),
        out_shape=jax.ShapeDtypeStruct((N, D), jnp.float32),
    )
    return tc23(agg2[0], agg2[1], c0, c1, h1, *w2, *w3)


# speedup vs baseline: 1.0884x; 1.0884x over previous
import functools

import jax
import jax.numpy as jnp
from jax import lax
from jax.experimental import pallas as pl
from jax.experimental.pallas import tpu as pltpu
from jax.experimental.pallas import tpu_sc as plsc

N = 10000
E = 320000
D = 128
NC = 2
NS = 16
NW = NC * NS
CH = 128
EPAD = 327680
ROWS = EPAD // CH
RW = ROWS // NW
NB = 2
G = RW // NB
NP = 10240
RPT = NP // NS


def _sc_agg_body(with_counts, *refs):
    if with_counts:
        (h_hbm, src_hbm, dst_hbm, zeros_hbm, zeros1_hbm, ones1_hbm,
         out_hbm, cnt_hbm,
         src_t, r0b, r1b, d0b, d1b, ones_v,
         acc_sh, cnt_sh,
         g0, g1, f0, f1, c0s, c1s) = refs
    else:
        (h_hbm, src_hbm, dst_hbm, zeros_hbm,
         out_hbm,
         src_t, r0b, r1b, d0b, d1b,
         acc_sh,
         g0, g1, f0, f1) = refs
    rows = [r0b, r1b]
    dstv = [d0b, d1b]
    gsem = [g0, g1]
    fsem = [f0, f1]
    if with_counts:
        csem = [c0s, c1s]

    c = lax.axis_index("c")
    s = lax.axis_index("s")
    wid = s * NC + c

    t0 = s * RPT
    pltpu.sync_copy(zeros_hbm, acc_sh.at[pl.ds(t0, RPT)])
    if with_counts:
        pltpu.sync_copy(zeros1_hbm, cnt_sh.at[pl.ds(t0, RPT)])
        pltpu.sync_copy(ones1_hbm, ones_v)
    plsc.subcore_barrier()

    base = wid * RW

    pltpu.sync_copy(src_hbm.at[pl.ds(base, RW)], src_t)

    for b in range(NB):
        off = (base + b) * CH
        pltpu.async_copy(dst_hbm.at[pl.ds(off, CH)], dstv[b], fsem[b])
        pltpu.async_copy(h_hbm.at[src_t.at[b]], rows[b], gsem[b])

    def grp(g, carry):
        i0 = g * NB
        for b in range(NB):
            pltpu.make_async_copy(h_hbm.at[pl.ds(0, CH)], rows[b],
                                  gsem[b]).wait()
            pltpu.make_async_copy(dst_hbm.at[pl.ds(0, CH)], dstv[b],
                                  fsem[b]).wait()
            if with_counts:
                pltpu.async_copy(ones_v, cnt_sh.at[dstv[b]], csem[b],
                                 add=True)
            pltpu.sync_copy(rows[b], acc_sh.at[dstv[b]], add=True)
            if with_counts:
                pltpu.make_async_copy(ones_v, cnt_sh.at[pl.ds(0, CH)],
                                      csem[b]).wait()

            @pl.when(g < G - 1)
            def _():
                nxt = i0 + b + NB
                off = pl.multiple_of((base + nxt) * CH, CH)
                pltpu.async_copy(dst_hbm.at[pl.ds(off, CH)], dstv[b], fsem[b])
                pltpu.async_copy(h_hbm.at[src_t.at[nxt]], rows[b], gsem[b])
        return carry

    lax.fori_loop(0, G, grp, 0)

    plsc.subcore_barrier()

    pltpu.sync_copy(acc_sh.at[pl.ds(t0, RPT)], out_hbm.at[c, pl.ds(t0, RPT)])
    if with_counts:
        pltpu.sync_copy(cnt_sh.at[pl.ds(t0, RPT)],
                        cnt_hbm.at[pl.ds(c * NP + t0, RPT)])


def _make_sc_agg(with_counts):
    mesh = plsc.VectorSubcoreMesh(core_axis_name="c", subcore_axis_name="s")
    out_type = [jax.ShapeDtypeStruct((NC, NP, D), jnp.float32)]
    if with_counts:
        out_type.append(jax.ShapeDtypeStruct((NC * NP,), jnp.float32))
    scratch = [pltpu.VMEM((RW, CH), jnp.int32)]
    scratch += [pltpu.VMEM((CH, D), jnp.float32)] * NB
    scratch += [pltpu.VMEM((CH,), jnp.int32)] * NB
    if with_counts:
        scratch += [pltpu.VMEM((CH,), jnp.float32)]
    scratch += [pltpu.VMEM_SHARED((NP, D), jnp.float32)]
    if with_counts:
        scratch += [pltpu.VMEM_SHARED((NP,), jnp.float32)]
    nsem = 3 * NB if with_counts else 2 * NB
    scratch += [pltpu.SemaphoreType.DMA] * nsem
    return pl.kernel(
        functools.partial(_sc_agg_body, with_counts),
        out_type=tuple(out_type) if with_counts else out_type[0],
        mesh=mesh,
        scratch_types=scratch,
    )


def _fastkan_block(mean, swa, swb, bwt, bias):
    ta = (mean + 1.0) * 0.5
    tb = (mean - 1.0) * 0.5
    ba = jnp.exp(-(ta * ta))
    bb = jnp.exp(-(tb * tb))
    sil = mean / (1.0 + jnp.exp(-mean))
    h = jnp.dot(ba, swa, preferred_element_type=jnp.float32)
    h += jnp.dot(bb, swb, preferred_element_type=jnp.float32)
    h += jnp.dot(sil, bwt, preferred_element_type=jnp.float32)
    return h + bias


def _layernorm_block(h):
    mu = jnp.mean(h, axis=-1, keepdims=True)
    var = jnp.mean((h - mu) ** 2, axis=-1, keepdims=True)
    return (h - mu) * lax.rsqrt(var + 1e-5)


def _tc_layer1_body(p0, p1, c0, c1, x, swa, swb, bwt, bias, out):
    agg = p0[...] + p1[...] + x[...]
    cnt = c0[...] + c1[...] + 1.0
    mean = agg / cnt
    h = _fastkan_block(mean, swa[...], swb[...], bwt[...], bias[...])
    out[...] = _layernorm_block(h)


def _tc_layer23_body(p0, p1, c0, c1, h1, swa2, swb2, bwt2, b2,
                     swa3, swb3, bwt3, b3, out):
    agg = p0[...] + p1[...] + h1[...]
    cnt = c0[...] + c1[...] + 1.0
    mean = agg / cnt
    h2 = _layernorm_block(
        _fastkan_block(mean, swa2[...], swb2[...], bwt2[...], b2[...]))
    o = _fastkan_block(h2, swa3[...], swb3[...], bwt3[...], b3[...])
    m = jnp.max(o, axis=-1, keepdims=True)
    lse = m + jnp.log(jnp.sum(jnp.exp(o - m), axis=-1, keepdims=True))
    out[...] = o - lse


_R = 1000


def _row_spec():
    return pl.BlockSpec((_R, D), lambda i: (i, 0))


def _cnt_spec():
    return pl.BlockSpec((_R, 1), lambda i: (i, 0))


def _w_spec():
    return pl.BlockSpec((D, D), lambda i: (0, 0))


def _b_spec():
    return pl.BlockSpec((1, D), lambda i: (0, 0))


def _split_w(sw, bw, sb, bb):
    swa = jnp.transpose(sw[:, 0::2])
    swb = jnp.transpose(sw[:, 1::2])
    bwt = jnp.transpose(bw)
    bias = (sb + bb).reshape(1, D)
    return swa, swb, bwt, bias


def kernel(x, edge_index, sw1, sb1, bw1, bb1, sw2, sb2, bw2, bb2,
           sw3, sb3, bw3, bb3):
    pad = EPAD - E
    pad_ar = jnp.arange(pad, dtype=jnp.int32)
    srcp = jnp.concatenate(
        [edge_index[0], pad_ar % N]).reshape(ROWS, CH)
    dstp = jnp.concatenate(
        [edge_index[1], N + pad_ar % (NP - N)])
    zeros = jnp.zeros((RPT, D), jnp.float32)
    zeros1 = jnp.zeros((RPT,), jnp.float32)
    ones1 = jnp.ones((CH,), jnp.float32)

    w1 = _split_w(sw1, bw1, sb1, bb1)
    w2 = _split_w(sw2, bw2, sb2, bb2)
    w3 = _split_w(sw3, bw3, sb3, bb3)

    agg1, cnt1d = _make_sc_agg(True)(x, srcp, dstp, zeros, zeros1, ones1)
    c0 = cnt1d[:N].reshape(N, 1)
    c1 = cnt1d[NP:NP + N].reshape(N, 1)

    tc1 = pl.pallas_call(
        _tc_layer1_body,
        grid=(N // _R,),
        in_specs=[_row_spec(), _row_spec(), _cnt_spec(), _cnt_spec(),
                  _row_spec(), _w_spec(), _w_spec(), _w_spec(), _b_spec()],
        out_specs=_row_spec(),
        out_shape=jax.ShapeDtypeStruct((N, D), jnp.float32),
    )
    h1 = tc1(agg1[0], agg1[1], c0, c1, x, *w1)

    agg2 = _make_sc_agg(False)(h1, srcp, dstp, zeros)

    tc23 = pl.pallas_call(
        _tc_layer23_body,
        grid=(N // _R,),
        in_specs=[_row_spec(), _row_spec(), _cnt_spec(), _cnt_spec(),
                  _row_spec(),
                  _w_spec(), _w_spec(), _w_spec(), _b_spec(),
                  _w_spec(), _w_spec(), _w_spec(), _b_spec()],
        out_specs=_row_spec(),
        out_shape=jax.ShapeDtypeStruct((N, D), jnp.float32),
    )
    return tc23(agg2[0], agg2[1], c0, c1, h1, *w2, *w3)


# speedup vs baseline: 1.1429x; 1.0501x over previous
import functools

import jax
import jax.numpy as jnp
from jax import lax
from jax.experimental import pallas as pl
from jax.experimental.pallas import tpu as pltpu
from jax.experimental.pallas import tpu_sc as plsc

N = 10000
E = 320000
D = 128
NC = 2
NS = 16
NW = NC * NS
CH = 120
EPAD = 322560
ROWS = EPAD // CH
RW = ROWS // NW
G = RW // 6
NP = 10240
RPT = NP // NS


def _sc_agg_body(with_counts, *refs):
    if with_counts:
        (h_hbm, src_hbm, dst_hbm, zeros_hbm, zeros1_hbm, ones1_hbm,
         out_hbm, cnt_hbm,
         r0b, r1b, r2b, sv0, sv1, sv2, sv3, sv4, sv5,
         d0b, d1b, d2b, d3b, d4b, d5b, ones_v,
         acc_sh, cnt_sh,
         g0, g1, g2, e0, e1, e2, e3, e4, e5, f0, f1, f2, f3, f4, f5,
         s0, s1, s2, c0s, c1s, c2s) = refs
    else:
        (h_hbm, src_hbm, dst_hbm, zeros_hbm,
         out_hbm,
         r0b, r1b, r2b, sv0, sv1, sv2, sv3, sv4, sv5,
         d0b, d1b, d2b, d3b, d4b, d5b,
         acc_sh,
         g0, g1, g2, e0, e1, e2, e3, e4, e5, f0, f1, f2, f3, f4, f5,
         s0, s1, s2) = refs
    rows = [r0b, r1b, r2b]
    srcv = [sv0, sv1, sv2, sv3, sv4, sv5]
    dstv = [d0b, d1b, d2b, d3b, d4b, d5b]
    gsem = [g0, g1, g2]
    esem = [e0, e1, e2, e3, e4, e5]
    fsem = [f0, f1, f2, f3, f4, f5]
    ssem = [s0, s1, s2]
    if with_counts:
        csem = [c0s, c1s, c2s]

    c = lax.axis_index("c")
    s = lax.axis_index("s")
    wid = s * NC + c

    t0 = s * RPT
    pltpu.sync_copy(zeros_hbm, acc_sh.at[pl.ds(t0, RPT)])
    if with_counts:
        pltpu.sync_copy(zeros1_hbm, cnt_sh.at[pl.ds(t0, RPT)])
        pltpu.sync_copy(ones1_hbm, ones_v)
    plsc.subcore_barrier()

    base = wid * RW

    def fetch_idx(islot, chunk):
        off = pl.multiple_of((base + chunk) * CH, 8)
        pltpu.async_copy(src_hbm.at[pl.ds(off, CH)], srcv[islot], esem[islot])
        pltpu.async_copy(dst_hbm.at[pl.ds(off, CH)], dstv[islot], fsem[islot])

    def fire_gather(rslot, islot):
        pltpu.make_async_copy(src_hbm.at[pl.ds(0, CH)], srcv[islot],
                              esem[islot]).wait()
        pltpu.async_copy(h_hbm.at[srcv[islot]], rows[rslot], gsem[rslot])

    def wait_scats(rslot):
        pltpu.make_async_copy(rows[rslot], acc_sh.at[pl.ds(0, CH)],
                              ssem[rslot]).wait()
        if with_counts:
            pltpu.make_async_copy(ones_v, cnt_sh.at[pl.ds(0, CH)],
                                  csem[rslot]).wait()

    for k in range(4):
        fetch_idx(k, k)
    fire_gather(0, 0)
    fire_gather(1, 1)

    def grp(g, carry):
        for b in range(6):
            rb = b % 3
            pltpu.make_async_copy(h_hbm.at[pl.ds(0, CH)], rows[rb],
                                  gsem[rb]).wait()
            pltpu.make_async_copy(dst_hbm.at[pl.ds(0, CH)], dstv[b],
                                  fsem[b]).wait()
            pltpu.async_copy(rows[rb], acc_sh.at[dstv[b]], ssem[rb], add=True)
            if with_counts:
                pltpu.async_copy(ones_v, cnt_sh.at[dstv[b]], csem[rb],
                                 add=True)

            rb2 = (rb + 2) % 3
            ib2 = (b + 2) % 6
            if b == 0:
                @pl.when(g > 0)
                def _():
                    wait_scats(rb2)
                fire_gather(rb2, ib2)
            elif b in (1, 2, 3):
                wait_scats(rb2)
                fire_gather(rb2, ib2)
            else:
                wait_scats(rb2)

                @pl.when(g < G - 1)
                def _():
                    fire_gather(rb2, ib2)

            ib4 = (b + 4) % 6
            if b in (0, 1):
                fetch_idx(ib4, 6 * g + b + 4)
            else:
                @pl.when(g < G - 1)
                def _():
                    fetch_idx(ib4, 6 * g + b + 4)
        return carry

    lax.fori_loop(0, G, grp, 0)

    wait_scats(2)

    plsc.subcore_barrier()

    pltpu.sync_copy(acc_sh.at[pl.ds(t0, RPT)], out_hbm.at[c, pl.ds(t0, RPT)])
    if with_counts:
        pltpu.sync_copy(cnt_sh.at[pl.ds(t0, RPT)],
                        cnt_hbm.at[pl.ds(c * NP + t0, RPT)])


def _make_sc_agg(with_counts):
    mesh = plsc.VectorSubcoreMesh(core_axis_name="c", subcore_axis_name="s")
    out_type = [jax.ShapeDtypeStruct((NC, NP, D), jnp.float32)]
    if with_counts:
        out_type.append(jax.ShapeDtypeStruct((NC * NP,), jnp.float32))
    scratch = []
    scratch += [pltpu.VMEM((CH, D), jnp.float32)] * 3
    scratch += [pltpu.VMEM((CH,), jnp.int32)] * 6
    scratch += [pltpu.VMEM((CH,), jnp.int32)] * 6
    if with_counts:
        scratch += [pltpu.VMEM((CH,), jnp.float32)]
    scratch += [pltpu.VMEM_SHARED((NP, D), jnp.float32)]
    if with_counts:
        scratch += [pltpu.VMEM_SHARED((NP,), jnp.float32)]
    nsem = (3 + 6 + 6 + 3) + (3 if with_counts else 0)
    scratch += [pltpu.SemaphoreType.DMA] * nsem
    return pl.kernel(
        functools.partial(_sc_agg_body, with_counts),
        out_type=tuple(out_type) if with_counts else out_type[0],
        mesh=mesh,
        scratch_types=scratch,
    )


def _fastkan_block(mean, swa, swb, bwt, bias):
    ta = (mean + 1.0) * 0.5
    tb = (mean - 1.0) * 0.5
    ba = jnp.exp(-(ta * ta))
    bb = jnp.exp(-(tb * tb))
    sil = mean / (1.0 + jnp.exp(-mean))
    h = jnp.dot(ba, swa, preferred_element_type=jnp.float32)
    h += jnp.dot(bb, swb, preferred_element_type=jnp.float32)
    h += jnp.dot(sil, bwt, preferred_element_type=jnp.float32)
    return h + bias


def _layernorm_block(h):
    mu = jnp.mean(h, axis=-1, keepdims=True)
    var = jnp.mean((h - mu) ** 2, axis=-1, keepdims=True)
    return (h - mu) * lax.rsqrt(var + 1e-5)


def _tc_layer1_body(p0, p1, c0, c1, x, swa, swb, bwt, bias, out):
    agg = p0[...] + p1[...] + x[...]
    cnt = c0[...] + c1[...] + 1.0
    mean = agg / cnt
    h = _fastkan_block(mean, swa[...], swb[...], bwt[...], bias[...])
    out[...] = _layernorm_block(h)


def _tc_layer23_body(p0, p1, c0, c1, h1, swa2, swb2, bwt2, b2,
                     swa3, swb3, bwt3, b3, out):
    agg = p0[...] + p1[...] + h1[...]
    cnt = c0[...] + c1[...] + 1.0
    mean = agg / cnt
    h2 = _layernorm_block(
        _fastkan_block(mean, swa2[...], swb2[...], bwt2[...], b2[...]))
    o = _fastkan_block(h2, swa3[...], swb3[...], bwt3[...], b3[...])
    m = jnp.max(o, axis=-1, keepdims=True)
    lse = m + jnp.log(jnp.sum(jnp.exp(o - m), axis=-1, keepdims=True))
    out[...] = o - lse


_R = 1000


def _row_spec():
    return pl.BlockSpec((_R, D), lambda i: (i, 0))


def _cnt_spec():
    return pl.BlockSpec((_R, 1), lambda i: (i, 0))


def _w_spec():
    return pl.BlockSpec((D, D), lambda i: (0, 0))


def _b_spec():
    return pl.BlockSpec((1, D), lambda i: (0, 0))


def _split_w(sw, bw, sb, bb):
    swa = jnp.transpose(sw[:, 0::2])
    swb = jnp.transpose(sw[:, 1::2])
    bwt = jnp.transpose(bw)
    bias = (sb + bb).reshape(1, D)
    return swa, swb, bwt, bias


def kernel(x, edge_index, sw1, sb1, bw1, bb1, sw2, sb2, bw2, bb2,
           sw3, sb3, bw3, bb3):
    pad = EPAD - E
    pad_ar = jnp.arange(pad, dtype=jnp.int32)
    srcp = jnp.concatenate([edge_index[0], pad_ar % N])
    dstp = jnp.concatenate([edge_index[1], N + pad_ar % (NP - N)])
    zeros = jnp.zeros((RPT, D), jnp.float32)
    zeros1 = jnp.zeros((RPT,), jnp.float32)
    ones1 = jnp.ones((CH,), jnp.float32)

    w1 = _split_w(sw1, bw1, sb1, bb1)
    w2 = _split_w(sw2, bw2, sb2, bb2)
    w3 = _split_w(sw3, bw3, sb3, bb3)

    agg1, cnt1d = _make_sc_agg(True)(x, srcp, dstp, zeros, zeros1, ones1)
    c0 = cnt1d[:N].reshape(N, 1)
    c1 = cnt1d[NP:NP + N].reshape(N, 1)

    tc1 = pl.pallas_call(
        _tc_layer1_body,
        grid=(N // _R,),
        in_specs=[_row_spec(), _row_spec(), _cnt_spec(), _cnt_spec(),
                  _row_spec(), _w_spec(), _w_spec(), _w_spec(), _b_spec()],
        out_specs=_row_spec(),
        out_shape=jax.ShapeDtypeStruct((N, D), jnp.float32),
    )
    h1 = tc1(agg1[0], agg1[1], c0, c1, x, *w1)

    agg2 = _make_sc_agg(False)(h1, srcp, dstp, zeros)

    tc23 = pl.pallas_call(
        _tc_layer23_body,
        grid=(N // _R,),
        in_specs=[_row_spec(), _row_spec(), _cnt_spec(), _cnt_spec(),
                  _row_spec(),
                  _w_spec(), _w_spec(), _w_spec(), _b_spec(),
                  _w_spec(), _w_spec(), _w_spec(), _b_spec()],
        out_specs=_row_spec(),
        out_shape=jax.ShapeDtypeStruct((N, D), jnp.float32),
    )
    return tc23(agg2[0], agg2[1], c0, c1, h1, *w2, *w3)


# speedup vs baseline: 1.1683x; 1.0222x over previous
import functools

import jax
import jax.numpy as jnp
from jax import lax
from jax.experimental import pallas as pl
from jax.experimental.pallas import tpu as pltpu
from jax.experimental.pallas import tpu_sc as plsc

N = 10000
E = 320000
D = 128
NC = 2
NS = 16
NW = NC * NS
CH = 120
EPAD = 322560
ROWS = EPAD // CH
RW = ROWS // NW
G = RW // 6
NP = 10240
RPT = NP // NS


def _sc_agg_body(with_counts, *refs):
    if with_counts:
        (h_hbm, src_hbm, dst_hbm, zeros_hbm, zeros1_hbm, ones1_hbm,
         out_hbm, cnt_hbm,
         r0b, r1b, r2b, sv0, sv1, sv2, sv3, sv4, sv5,
         d0b, d1b, d2b, d3b, d4b, d5b, ones_v,
         acc_sh, cnt_sh,
         g0, g1, g2, e0, e1, e2, e3, e4, e5, f0, f1, f2, f3, f4, f5,
         s0, s1, s2, c0s, c1s, c2s) = refs
    else:
        (h_hbm, src_hbm, dst_hbm, zeros_hbm,
         out_hbm,
         r0b, r1b, r2b, sv0, sv1, sv2, sv3, sv4, sv5,
         d0b, d1b, d2b, d3b, d4b, d5b,
         acc_sh,
         g0, g1, g2, e0, e1, e2, e3, e4, e5, f0, f1, f2, f3, f4, f5,
         s0, s1, s2) = refs
    rows = [r0b, r1b, r2b]
    srcv = [sv0, sv1, sv2, sv3, sv4, sv5]
    dstv = [d0b, d1b, d2b, d3b, d4b, d5b]
    gsem = [g0, g1, g2]
    esem = [e0, e1, e2, e3, e4, e5]
    fsem = [f0, f1, f2, f3, f4, f5]
    ssem = [s0, s1, s2]
    if with_counts:
        csem = [c0s, c1s, c2s]

    c = lax.axis_index("c")
    s = lax.axis_index("s")
    wid = s * NC + c

    t0 = s * RPT
    pltpu.sync_copy(zeros_hbm, acc_sh.at[pl.ds(t0, RPT)])
    if with_counts:
        pltpu.sync_copy(zeros1_hbm, cnt_sh.at[pl.ds(t0, RPT)])
        pltpu.sync_copy(ones1_hbm, ones_v)
    plsc.subcore_barrier()

    base = wid * RW

    def fetch_idx(islot, chunk):
        off = pl.multiple_of((base + chunk) * CH, 8)
        pltpu.async_copy(src_hbm.at[pl.ds(off, CH)], srcv[islot], esem[islot])
        pltpu.async_copy(dst_hbm.at[pl.ds(off, CH)], dstv[islot], fsem[islot])

    def fire_gather(rslot, islot):
        pltpu.make_async_copy(src_hbm.at[pl.ds(0, CH)], srcv[islot],
                              esem[islot]).wait()
        pltpu.async_copy(h_hbm.at[srcv[islot]], rows[rslot], gsem[rslot])

    def wait_scats(rslot):
        pltpu.make_async_copy(rows[rslot], acc_sh.at[pl.ds(0, CH)],
                              ssem[rslot]).wait()
        if with_counts:
            pltpu.make_async_copy(ones_v, cnt_sh.at[pl.ds(0, CH)],
                                  csem[rslot]).wait()

    for k in range(4):
        fetch_idx(k, k)
    fire_gather(0, 0)
    fire_gather(1, 1)

    def grp(g, carry):
        for b in range(6):
            rb = b % 3
            pltpu.make_async_copy(h_hbm.at[pl.ds(0, CH)], rows[rb],
                                  gsem[rb]).wait()
            pltpu.make_async_copy(dst_hbm.at[pl.ds(0, CH)], dstv[b],
                                  fsem[b]).wait()
            pltpu.async_copy(rows[rb], acc_sh.at[dstv[b]], ssem[rb], add=True)
            if with_counts:
                pltpu.async_copy(ones_v, cnt_sh.at[dstv[b]], csem[rb],
                                 add=True)

            rb2 = (rb + 2) % 3
            ib2 = (b + 2) % 6
            if b == 0:
                @pl.when(g > 0)
                def _():
                    wait_scats(rb2)
                fire_gather(rb2, ib2)
            elif b in (1, 2, 3):
                wait_scats(rb2)
                fire_gather(rb2, ib2)
            else:
                wait_scats(rb2)

                @pl.when(g < G - 1)
                def _():
                    fire_gather(rb2, ib2)

            ib4 = (b + 4) % 6
            if b in (0, 1):
                fetch_idx(ib4, 6 * g + b + 4)
            else:
                @pl.when(g < G - 1)
                def _():
                    fetch_idx(ib4, 6 * g + b + 4)
        return carry

    lax.fori_loop(0, G, grp, 0)

    wait_scats(2)

    plsc.subcore_barrier()

    pltpu.sync_copy(acc_sh.at[pl.ds(t0, RPT)], out_hbm.at[c, pl.ds(t0, RPT)])
    if with_counts:
        pltpu.sync_copy(cnt_sh.at[pl.ds(t0, RPT)],
                        cnt_hbm.at[pl.ds(c * NP + t0, RPT)])


def _make_sc_agg(with_counts):
    mesh = plsc.VectorSubcoreMesh(core_axis_name="c", subcore_axis_name="s")
    out_type = [jax.ShapeDtypeStruct((NC, NP, D), jnp.float32)]
    if with_counts:
        out_type.append(jax.ShapeDtypeStruct((NC * NP,), jnp.float32))
    scratch = []
    scratch += [pltpu.VMEM((CH, D), jnp.float32)] * 3
    scratch += [pltpu.VMEM((CH,), jnp.int32)] * 6
    scratch += [pltpu.VMEM((CH,), jnp.int32)] * 6
    if with_counts:
        scratch += [pltpu.VMEM((CH,), jnp.float32)]
    scratch += [pltpu.VMEM_SHARED((NP, D), jnp.float32)]
    if with_counts:
        scratch += [pltpu.VMEM_SHARED((NP,), jnp.float32)]
    nsem = (3 + 6 + 6 + 3) + (3 if with_counts else 0)
    scratch += [pltpu.SemaphoreType.DMA] * nsem
    return pl.kernel(
        functools.partial(_sc_agg_body, with_counts),
        out_type=tuple(out_type) if with_counts else out_type[0],
        mesh=mesh,
        scratch_types=scratch,
    )


def _fastkan_block(mean, swa, swb, bwt, bias):
    ta = (mean + 1.0) * 0.5
    tb = (mean - 1.0) * 0.5
    ba = jnp.exp(-(ta * ta))
    bb = jnp.exp(-(tb * tb))
    sil = mean / (1.0 + jnp.exp(-mean))
    h = jnp.dot(ba, swa, preferred_element_type=jnp.float32)
    h += jnp.dot(bb, swb, preferred_element_type=jnp.float32)
    h += jnp.dot(sil, bwt, preferred_element_type=jnp.float32)
    return h + bias


def _layernorm_block(h):
    mu = jnp.mean(h, axis=-1, keepdims=True)
    var = jnp.mean((h - mu) ** 2, axis=-1, keepdims=True)
    return (h - mu) * lax.rsqrt(var + 1e-5)


def _tc_layer1_body(p0, p1, c0, c1, x, swa, swb, bwt, bias, out):
    agg = p0[...] + p1[...] + x[...]
    cnt = c0[...] + c1[...] + 1.0
    mean = agg / cnt
    h = _fastkan_block(mean, swa[...], swb[...], bwt[...], bias[...])
    out[...] = _layernorm_block(h)


def _tc_layer23_body(p0, p1, c0, c1, h1, swa2, swb2, bwt2, b2,
                     swa3, swb3, bwt3, b3, out):
    agg = p0[...] + p1[...] + h1[...]
    cnt = c0[...] + c1[...] + 1.0
    mean = agg / cnt
    h2 = _layernorm_block(
        _fastkan_block(mean, swa2[...], swb2[...], bwt2[...], b2[...]))
    o = _fastkan_block(h2, swa3[...], swb3[...], bwt3[...], b3[...])
    m = jnp.max(o, axis=-1, keepdims=True)
    lse = m + jnp.log(jnp.sum(jnp.exp(o - m), axis=-1, keepdims=True))
    out[...] = o - lse


_R = 2000


def _row_spec():
    return pl.BlockSpec((_R, D), lambda i: (i, 0))


def _cnt_spec():
    return pl.BlockSpec((_R, 1), lambda i: (i, 0))


def _w_spec():
    return pl.BlockSpec((D, D), lambda i: (0, 0))


def _b_spec():
    return pl.BlockSpec((1, D), lambda i: (0, 0))


def _split_w(sw, bw, sb, bb):
    swa = jnp.transpose(sw[:, 0::2])
    swb = jnp.transpose(sw[:, 1::2])
    bwt = jnp.transpose(bw)
    bias = (sb + bb).reshape(1, D)
    return swa, swb, bwt, bias


def kernel(x, edge_index, sw1, sb1, bw1, bb1, sw2, sb2, bw2, bb2,
           sw3, sb3, bw3, bb3):
    pad = EPAD - E
    pad_ar = jnp.arange(pad, dtype=jnp.int32)
    srcp = jnp.concatenate([edge_index[0], pad_ar % N])
    dstp = jnp.concatenate([edge_index[1], N + pad_ar % (NP - N)])
    zeros = jnp.zeros((RPT, D), jnp.float32)
    zeros1 = jnp.zeros((RPT,), jnp.float32)
    ones1 = jnp.ones((CH,), jnp.float32)

    w1 = _split_w(sw1, bw1, sb1, bb1)
    w2 = _split_w(sw2, bw2, sb2, bb2)
    w3 = _split_w(sw3, bw3, sb3, bb3)

    agg1, cnt1d = _make_sc_agg(True)(x, srcp, dstp, zeros, zeros1, ones1)
    c0 = cnt1d[:N].reshape(N, 1)
    c1 = cnt1d[NP:NP + N].reshape(N, 1)

    tc1 = pl.pallas_call(
        _tc_layer1_body,
        grid=(N // _R,),
        in_specs=[_row_spec(), _row_spec(), _cnt_spec(), _cnt_spec(),
                  _row_spec(), _w_spec(), _w_spec(), _w_spec(), _b_spec()],
        out_specs=_row_spec(),
        out_shape=jax.ShapeDtypeStruct((N, D), jnp.float32),
    )
    h1 = tc1(agg1[0], agg1[1], c0, c1, x, *w1)

    agg2 = _make_sc_agg(False)(h1, srcp, dstp, zeros)

    tc23 = pl.pallas_call(
        _tc_layer23_body,
        grid=(N // _R,),
        in_specs=[_row_spec(), _row_spec(), _cnt_spec(), _cnt_spec(),
                  _row_spec(),
                  _w_spec(), _w_spec(), _w_spec(), _b_spec(),
                  _w_spec(), _w_spec(), _w_spec(), _b_spec()],
        out_specs=_row_spec(),
        out_shape=jax.ShapeDtypeStruct((N, D), jnp.float32),
    )
    return tc23(agg2[0], agg2[1], c0, c1, h1, *w2, *w3)
